# baseline (speedup 1.0000x reference)
import jax
import jax.numpy as jnp
from jax import lax
from jax.experimental import pallas as pl
from jax.experimental.pallas import tpu as pltpu
from jax.experimental.pallas import tpu_sc as plsc

EPS = 1e-8
NQ = 512
QG = 128


def _knn_body(p1t_ref, p2_ref, idx_ref, w_ref):
    b = pl.program_id(0)
    p1t = p1t_ref[0]
    p2t = p2_ref[0]
    m = p1t.shape[1]

    p1sq = jnp.sum(p1t * p1t, axis=0, keepdims=True)
    p2sq = (p2t[:, 0:1] * p2t[:, 0:1] + p2t[:, 1:2] * p2t[:, 1:2]
            + p2t[:, 2:3] * p2t[:, 2:3])
    cross = lax.dot_general(p2t, p1t, (((1,), (0,)), ((), ())),
                            preferred_element_type=jnp.float32)
    d2 = jnp.maximum(p2sq + p1sq - 2.0 * cross, 0.0)

    iota = lax.broadcasted_iota(jnp.int32, d2.shape, 1).astype(jnp.float32)
    d = d2
    imns, recips = [], []
    for k in range(3):
        mn = jnp.min(d, axis=1, keepdims=True)
        imn = jnp.min(jnp.where(d == mn, iota, float(m)), axis=1,
                      keepdims=True)
        imns.append(imn)
        recips.append(1.0 / (mn + EPS))
        if k < 2:
            d = jnp.where(iota == imn, jnp.inf, d)

    norm = recips[0] + recips[1] + recips[2]
    gbase = (b * m).astype(jnp.float32)
    idx_ref[0] = jnp.concatenate(
        [gbase + imns[0], gbase + imns[1], gbase + imns[2]],
        axis=1).astype(jnp.int32)
    w_ref[0] = jnp.concatenate(
        [recips[0] / norm, recips[1] / norm, recips[2] / norm], axis=1)


def _transpose_body(x1_ref, xt_ref):
    xt_ref[0] = x1_ref[0].T


def _sc_interp_kernel(C, G, n_groups_b):
    info = plsc.get_sparse_core_info()
    nc, ns = info.num_cores, info.num_subcores
    nw = nc * ns
    g_per_w = G // nw
    n_chunks = QG // 16
    mesh = plsc.VectorSubcoreMesh(core_axis_name="c", subcore_axis_name="s")

    def body(xt_hbm, idx_hbm, w_hbm, out_hbm, idx_v, w_v, rows_a, rows_b,
             acc_v, sem_a, sem_b):
        wid = lax.axis_index("s") * nc + lax.axis_index("c")
        lane = lax.broadcasted_iota(jnp.int32, (16,), 0)
        rid0 = lane * 3
        rid1 = rid0 + 1
        rid2 = rid0 + 2
        bufs = (rows_a, rows_b)
        sems = (sem_a, sem_b)

        def group(i, carry):
            g = wid * g_per_w + i
            b = g // n_groups_b
            qo = (g - b * n_groups_b) * QG
            base = g * (QG * 3)
            pltpu.sync_copy(idx_hbm.at[pl.ds(base, QG * 3)], idx_v)
            pltpu.sync_copy(w_hbm.at[pl.ds(base, QG * 3)], w_v)

            def start(s, buf_i):
                return pltpu.async_copy(
                    xt_hbm.at[idx_v.at[pl.ds(s * 48, 48)]],
                    bufs[buf_i].at[:, pl.ds(0, C)], sems[buf_i])

            def wait_buf(buf_i):
                pltpu.make_async_copy(
                    xt_hbm.at[idx_v.at[pl.ds(0, 48)]],
                    bufs[buf_i].at[:, pl.ds(0, C)], sems[buf_i]).wait()

            def compute(s, buf_i):
                rows = bufs[buf_i]
                soff = s * 48
                qoff = s * 16

                @plsc.parallel_loop(0, 16, unroll=2)
                def query_body(q):
                    wbase = soff + q * 3
                    w0 = plsc.load_gather(w_v, [lane * 0 + wbase])
                    w1 = plsc.load_gather(w_v, [lane * 0 + wbase + 1])
                    w2 = plsc.load_gather(w_v, [lane * 0 + wbase + 2])
                    qvec = jnp.zeros((16,), jnp.int32) + (qoff + q)
                    r0 = q * 3
                    for cb in range(C // 16):
                        c0 = cb * 16
                        cvec = lane + c0
                        g0 = rows[r0, pl.ds(c0, 16)]
                        g1 = rows[r0 + 1, pl.ds(c0, 16)]
                        g2 = rows[r0 + 2, pl.ds(c0, 16)]
                        plsc.store_scatter(acc_v, [cvec, qvec],
                                           w0 * g0 + w1 * g1 + w2 * g2)

            start(0, 0)
            start(1, 1)

            def pair(s2, carry2):
                s0 = 2 * s2
                not_last = s2 < (n_chunks // 2 - 1)
                wait_buf(0)
                compute(s0, 0)

                @pl.when(not_last)
                def _():
                    start(s0 + 2, 0)

                wait_buf(1)
                compute(s0 + 1, 1)

                @pl.when(not_last)
                def _():
                    start(s0 + 3, 1)

                return carry2

            lax.fori_loop(0, n_chunks // 2, pair, 0)
            pltpu.sync_copy(acc_v, out_hbm.at[b, :, pl.ds(qo, QG)])
            return carry

        lax.fori_loop(0, g_per_w, group, 0)

    return mesh, body


def kernel(p1, x1, p2):
    B, M, _ = p1.shape
    C = x1.shape[1]
    N = p2.shape[1]
    p1t = jnp.transpose(p1, (0, 2, 1))

    idx3, w3 = pl.pallas_call(
        _knn_body,
        grid=(B, N // NQ),
        in_specs=[
            pl.BlockSpec((1, 3, M), lambda b, q: (b, 0, 0)),
            pl.BlockSpec((1, NQ, 3), lambda b, q: (b, q, 0)),
        ],
        out_specs=[
            pl.BlockSpec((1, NQ, 3), lambda b, q: (b, q, 0)),
            pl.BlockSpec((1, NQ, 3), lambda b, q: (b, q, 0)),
        ],
        out_shape=[
            jax.ShapeDtypeStruct((B, N, 3), jnp.int32),
            jax.ShapeDtypeStruct((B, N, 3), jnp.float32),
        ],
    )(p1t, p2)

    x1t = pl.pallas_call(
        _transpose_body,
        grid=(B,),
        in_specs=[pl.BlockSpec((1, C, M), lambda b: (b, 0, 0))],
        out_specs=pl.BlockSpec((1, M, C), lambda b: (b, 0, 0)),
        out_shape=jax.ShapeDtypeStruct((B, M, C), jnp.float32),
    )(x1)

    G = (B * N) // QG
    n_groups_b = N // QG
    idx_f = idx3.reshape(B * N * 3)
    w_f = w3.reshape(B * N * 3)
    xt_flat = x1t.reshape(B * M, C)

    mesh, body = _sc_interp_kernel(C, G, n_groups_b)
    out = pl.kernel(
        body,
        mesh=mesh,
        out_type=jax.ShapeDtypeStruct((B, C, N), jnp.float32),
        scratch_types=[
            pltpu.VMEM((QG * 3,), jnp.int32),
            pltpu.VMEM((QG * 3,), jnp.float32),
            pltpu.VMEM((48, C + 1), jnp.float32),
            pltpu.VMEM((48, C + 1), jnp.float32),
            pltpu.VMEM((C, QG), jnp.float32),
            pltpu.SemaphoreType.DMA,
            pltpu.SemaphoreType.DMA,
        ],
        compiler_params=pltpu.CompilerParams(needs_layout_passes=False),
    )(xt_flat, idx_f, w_f)
    return out


# speedup vs baseline: 1.0656x; 1.0656x over previous
import jax
import jax.numpy as jnp
from jax import lax
from jax.experimental import pallas as pl
from jax.experimental.pallas import tpu as pltpu
from jax.experimental.pallas import tpu_sc as plsc

EPS = 1e-8
NQ = 512
QG = 128


def _knn_body(p1t_ref, p2_ref, idx_ref, w_ref):
    b = pl.program_id(0)
    p1t = p1t_ref[0]
    p2t = p2_ref[0]
    m = p1t.shape[1]

    p1sq = jnp.sum(p1t * p1t, axis=0, keepdims=True)
    p2sq = (p2t[:, 0:1] * p2t[:, 0:1] + p2t[:, 1:2] * p2t[:, 1:2]
            + p2t[:, 2:3] * p2t[:, 2:3])
    cross = lax.dot_general(p2t, p1t, (((1,), (0,)), ((), ())),
                            preferred_element_type=jnp.float32)
    d2 = jnp.maximum(p2sq + p1sq - 2.0 * cross, 0.0)

    iota = lax.broadcasted_iota(jnp.int32, d2.shape, 1).astype(jnp.float32)
    d = d2
    imns, recips = [], []
    for k in range(3):
        mn = jnp.min(d, axis=1, keepdims=True)
        imn = jnp.min(jnp.where(d == mn, iota, float(m)), axis=1,
                      keepdims=True)
        imns.append(imn)
        recips.append(1.0 / (mn + EPS))
        if k < 2:
            d = jnp.where(iota == imn, jnp.inf, d)

    norm = recips[0] + recips[1] + recips[2]
    gbase = (b * m).astype(jnp.float32)
    idx_ref[0] = jnp.concatenate(
        [gbase + imns[0], gbase + imns[1], gbase + imns[2]],
        axis=1).astype(jnp.int32)
    w_ref[0] = jnp.concatenate(
        [recips[0] / norm, recips[1] / norm, recips[2] / norm], axis=1)


def _transpose_body(x1_ref, xt_ref):
    xt_ref[0] = x1_ref[0].T


def _sc_interp_kernel(C, G, n_groups_b):
    info = plsc.get_sparse_core_info()
    nc, ns = info.num_cores, info.num_subcores
    nw = nc * ns
    g_per_w = G // nw
    n_chunks = QG // 16
    mesh = plsc.VectorSubcoreMesh(core_axis_name="c", subcore_axis_name="s")

    def body(xt_hbm, idx_hbm, w_hbm, out_hbm, idx_v, w_v, rows_a, rows_b,
             acc_v, sem_a, sem_b):
        wid = lax.axis_index("s") * nc + lax.axis_index("c")
        lane = lax.broadcasted_iota(jnp.int32, (16,), 0)
        rid0 = lane * 3
        rid1 = rid0 + 1
        rid2 = rid0 + 2
        bufs = (rows_a, rows_b)
        sems = (sem_a, sem_b)

        def group(i, carry):
            g = wid * g_per_w + i
            b = g // n_groups_b
            qo = (g - b * n_groups_b) * QG
            base = g * (QG * 3)
            pltpu.sync_copy(idx_hbm.at[pl.ds(base, QG * 3)], idx_v)
            pltpu.sync_copy(w_hbm.at[pl.ds(base, QG * 3)], w_v)

            def start(s):
                return pltpu.async_copy(
                    xt_hbm.at[idx_v.at[pl.ds(s * 48, 48)]],
                    bufs[s % 2].at[:, pl.ds(0, C)], sems[s % 2])

            pending = start(0)
            for s in range(n_chunks):
                nxt = start(s + 1) if s + 1 < n_chunks else None
                pending.wait()
                rows = bufs[s % 2]
                soff = s * 48
                qoff = s * 16

                @plsc.parallel_loop(0, 16, unroll=1)
                def query_body(q, rows=rows, soff=soff, qoff=qoff):
                    wbase = soff + q * 3
                    w0 = plsc.load_gather(w_v, [lane * 0 + wbase])
                    w1 = plsc.load_gather(w_v, [lane * 0 + wbase + 1])
                    w2 = plsc.load_gather(w_v, [lane * 0 + wbase + 2])
                    qvec = jnp.zeros((16,), jnp.int32) + (qoff + q)
                    r0 = q * 3
                    for cb in range(C // 16):
                        c0 = cb * 16
                        cvec = lane + c0
                        g0 = rows[r0, pl.ds(c0, 16)]
                        g1 = rows[r0 + 1, pl.ds(c0, 16)]
                        g2 = rows[r0 + 2, pl.ds(c0, 16)]
                        plsc.store_scatter(acc_v, [cvec, qvec],
                                           w0 * g0 + w1 * g1 + w2 * g2)
                pending = nxt
            pltpu.sync_copy(acc_v, out_hbm.at[b, :, pl.ds(qo, QG)])
            return carry

        lax.fori_loop(0, g_per_w, group, 0)

    return mesh, body


def kernel(p1, x1, p2):
    B, M, _ = p1.shape
    C = x1.shape[1]
    N = p2.shape[1]
    p1t = jnp.transpose(p1, (0, 2, 1))

    idx3, w3 = pl.pallas_call(
        _knn_body,
        grid=(B, N // NQ),
        in_specs=[
            pl.BlockSpec((1, 3, M), lambda b, q: (b, 0, 0)),
            pl.BlockSpec((1, NQ, 3), lambda b, q: (b, q, 0)),
        ],
        out_specs=[
            pl.BlockSpec((1, NQ, 3), lambda b, q: (b, q, 0)),
            pl.BlockSpec((1, NQ, 3), lambda b, q: (b, q, 0)),
        ],
        out_shape=[
            jax.ShapeDtypeStruct((B, N, 3), jnp.int32),
            jax.ShapeDtypeStruct((B, N, 3), jnp.float32),
        ],
    )(p1t, p2)

    x1t = pl.pallas_call(
        _transpose_body,
        grid=(B,),
        in_specs=[pl.BlockSpec((1, C, M), lambda b: (b, 0, 0))],
        out_specs=pl.BlockSpec((1, M, C), lambda b: (b, 0, 0)),
        out_shape=jax.ShapeDtypeStruct((B, M, C), jnp.float32),
    )(x1)

    G = (B * N) // QG
    n_groups_b = N // QG
    idx_f = idx3.reshape(B * N * 3)
    w_f = w3.reshape(B * N * 3)
    xt_flat = x1t.reshape(B * M, C)

    mesh, body = _sc_interp_kernel(C, G, n_groups_b)
    out = pl.kernel(
        body,
        mesh=mesh,
        out_type=jax.ShapeDtypeStruct((B, C, N), jnp.float32),
        scratch_types=[
            pltpu.VMEM((QG * 3,), jnp.int32),
            pltpu.VMEM((QG * 3,), jnp.float32),
            pltpu.VMEM((48, C + 1), jnp.float32),
            pltpu.VMEM((48, C + 1), jnp.float32),
            pltpu.VMEM((C, QG), jnp.float32),
            pltpu.SemaphoreType.DMA,
            pltpu.SemaphoreType.DMA,
        ],
        compiler_params=pltpu.CompilerParams(needs_layout_passes=False),
    )(xt_flat, idx_f, w_f)
    return out
